# initial kernel scaffold (unmeasured)
import jax
import jax.numpy as jnp
from jax import lax
from jax.experimental import pallas as pl
from jax.experimental.pallas import tpu as pltpu

M = 8192
N = 4096
TM = 512


def kernel(x, w_mat):
    K = x.shape[1]

    def body(x_hbm, w_ref, out_hbm, recv_a, recv_b, recv_c,
             x_vmem, acc_vmem, add_vmem, local_sems, send_sems, recv_sems):
        my = lax.axis_index("i")
        z = my // 4
        j = my % 4
        y = j // 2
        xb = ((j + 1) // 2) % 2
        pz = my ^ 4
        py = 4 * z + (3 - j)
        px = my ^ 1

        bsem = pltpu.get_barrier_semaphore()
        for p in (px, py, pz):
            pl.semaphore_signal(bsem, inc=1, device_id=(p,),
                                device_id_type=pl.DeviceIdType.MESH)
        pl.semaphore_wait(bsem, 3)

        def gemm_step(i, carry):
            ld = pltpu.make_async_copy(
                x_hbm.at[pl.ds(i * TM, TM)], x_vmem, local_sems.at[0])
            ld.start()
            ld.wait()
            acc_vmem[...] = jnp.dot(x_vmem[...], w_ref[...],
                                    preferred_element_type=jnp.float32)
            st = pltpu.make_async_copy(
                acc_vmem, out_hbm.at[pl.ds(i * TM, TM)], local_sems.at[1])
            st.start()
            st.wait()
            return carry

        lax.fori_loop(0, M // TM, gemm_step, 0)

        def exchange(sidx, partner, send_base, nrows, recv_buf):
            rdma = pltpu.make_async_remote_copy(
                src_ref=out_hbm.at[pl.ds(send_base, nrows)],
                dst_ref=recv_buf,
                send_sem=send_sems.at[sidx],
                recv_sem=recv_sems.at[sidx],
                device_id=(partner,),
                device_id_type=pl.DeviceIdType.MESH,
            )
            rdma.start()
            rdma.wait()

        def add_from(recv_buf, keep_base, nrows, fuse_silu):
            def step(i, carry):
                r0 = keep_base + i * TM
                c1 = pltpu.make_async_copy(
                    out_hbm.at[pl.ds(r0, TM)], acc_vmem, local_sems.at[0])
                c2 = pltpu.make_async_copy(
                    recv_buf.at[pl.ds(i * TM, TM)], add_vmem, local_sems.at[1])
                c1.start()
                c2.start()
                c1.wait()
                c2.wait()
                s = acc_vmem[...] + add_vmem[...]
                if fuse_silu:
                    s = s * jax.nn.sigmoid(s)
                acc_vmem[...] = s
                st = pltpu.make_async_copy(
                    acc_vmem, out_hbm.at[pl.ds(r0, TM)], local_sems.at[0])
                st.start()
                st.wait()
                return carry

            lax.fori_loop(0, nrows // TM, step, 0)

        half_a = z * 4096
        base_b = half_a + 2048 * y
        base_c = base_b + 1024 * xb

        exchange(0, pz, (1 - z) * 4096, 4096, recv_a)
        add_from(recv_a, half_a, 4096, False)
        exchange(1, py, half_a + (1 - y) * 2048, 2048, recv_b)
        add_from(recv_b, base_b, 2048, False)
        exchange(2, px, base_b + (1 - xb) * 1024, 1024, recv_c)
        add_from(recv_c, base_c, 1024, True)

        def ag(sidx, partner, my_base, nrows):
            rdma = pltpu.make_async_remote_copy(
                src_ref=out_hbm.at[pl.ds(my_base, nrows)],
                dst_ref=out_hbm.at[pl.ds(my_base, nrows)],
                send_sem=send_sems.at[sidx],
                recv_sem=recv_sems.at[sidx],
                device_id=(partner,),
                device_id_type=pl.DeviceIdType.MESH,
            )
            rdma.start()
            rdma.wait()

        ag(3, px, base_c, 1024)
        ag(4, py, base_b, 2048)
        ag(5, pz, half_a, 4096)

    outs = pl.pallas_call(
        body,
        out_shape=(
            jax.ShapeDtypeStruct((M, N), jnp.float32),
        ),
        in_specs=[
            pl.BlockSpec(memory_space=pltpu.MemorySpace.ANY),
            pl.BlockSpec(memory_space=pltpu.MemorySpace.VMEM),
        ],
        out_specs=(pl.BlockSpec(memory_space=pltpu.MemorySpace.ANY),),
        scratch_shapes=[
            pltpu.HBM((4096, N), jnp.float32),
            pltpu.HBM((2048, N), jnp.float32),
            pltpu.HBM((1024, N), jnp.float32),
            pltpu.VMEM((TM, K), jnp.float32),
            pltpu.VMEM((TM, N), jnp.float32),
            pltpu.VMEM((TM, N), jnp.float32),
            pltpu.SemaphoreType.DMA((2,)),
            pltpu.SemaphoreType.DMA((6,)),
            pltpu.SemaphoreType.DMA((6,)),
        ],
        compiler_params=pltpu.CompilerParams(collective_id=0),
    )(x, w_mat)
    return outs[0]


# baseline (device time: 2917619 ns/iter reference)
import jax
import jax.numpy as jnp
from jax import lax
from jax.experimental import pallas as pl
from jax.experimental.pallas import tpu as pltpu

M = 8192
N = 4096
TM = 512


def kernel(x, w_mat):
    K = x.shape[1]

    def body(x_hbm, w_ref, out_hbm, recv_a, recv_b, recv_c,
             x_vmem, acc_vmem, add_vmem, local_sems, send_sems, recv_sems):
        my = lax.axis_index("i")
        z = my // 4
        j = my % 4
        y = j // 2
        xb = ((j + 1) // 2) % 2
        pz = my ^ 4
        py = 4 * z + (3 - j)
        px = my ^ 1

        bsem = pltpu.get_barrier_semaphore()
        for p in (px, py, pz):
            pl.semaphore_signal(bsem, inc=1, device_id=(p,),
                                device_id_type=pl.DeviceIdType.MESH)
        pl.semaphore_wait(bsem, 3)

        def gemm_step(i, carry):
            ld = pltpu.make_async_copy(
                x_hbm.at[pl.ds(i * TM, TM)], x_vmem, local_sems.at[0])
            ld.start()
            ld.wait()
            acc_vmem[...] = jnp.dot(x_vmem[...], w_ref[...],
                                    preferred_element_type=jnp.float32)
            st = pltpu.make_async_copy(
                acc_vmem, out_hbm.at[pl.ds(i * TM, TM)], local_sems.at[1])
            st.start()
            st.wait()
            return carry

        lax.fori_loop(0, M // TM, gemm_step, 0)

        def exchange(sidx, partner, send_base, nrows, recv_buf):
            rdma = pltpu.make_async_remote_copy(
                src_ref=out_hbm.at[pl.ds(send_base, nrows)],
                dst_ref=recv_buf,
                send_sem=send_sems.at[sidx],
                recv_sem=recv_sems.at[sidx],
                device_id=(partner,),
                device_id_type=pl.DeviceIdType.MESH,
            )
            rdma.start()
            rdma.wait()

        def add_from(recv_buf, keep_base, nrows, fuse_silu):
            def step(i, carry):
                r0 = keep_base + i * TM
                c1 = pltpu.make_async_copy(
                    out_hbm.at[pl.ds(r0, TM)], acc_vmem, local_sems.at[0])
                c2 = pltpu.make_async_copy(
                    recv_buf.at[pl.ds(i * TM, TM)], add_vmem, local_sems.at[1])
                c1.start()
                c2.start()
                c1.wait()
                c2.wait()
                s = acc_vmem[...] + add_vmem[...]
                if fuse_silu:
                    s = s * jax.nn.sigmoid(s)
                acc_vmem[...] = s
                st = pltpu.make_async_copy(
                    acc_vmem, out_hbm.at[pl.ds(r0, TM)], local_sems.at[0])
                st.start()
                st.wait()
                return carry

            lax.fori_loop(0, nrows // TM, step, 0)

        half_a = z * 4096
        base_b = half_a + 2048 * y
        base_c = base_b + 1024 * xb

        exchange(0, pz, (1 - z) * 4096, 4096, recv_a)
        add_from(recv_a, half_a, 4096, False)
        exchange(1, py, half_a + (1 - y) * 2048, 2048, recv_b)
        add_from(recv_b, base_b, 2048, False)
        exchange(2, px, base_b + (1 - xb) * 1024, 1024, recv_c)
        add_from(recv_c, base_c, 1024, True)

        def ag(sidx, partner, my_base, nrows):
            rdma = pltpu.make_async_remote_copy(
                src_ref=out_hbm.at[pl.ds(my_base, nrows)],
                dst_ref=out_hbm.at[pl.ds(my_base, nrows)],
                send_sem=send_sems.at[sidx],
                recv_sem=recv_sems.at[sidx],
                device_id=(partner,),
                device_id_type=pl.DeviceIdType.MESH,
            )
            rdma.start()
            rdma.wait()

        ag(3, px, base_c, 1024)
        ag(4, py, base_b, 2048)
        ag(5, pz, half_a, 4096)

    outs = pl.pallas_call(
        body,
        out_shape=(
            jax.ShapeDtypeStruct((M, N), jnp.float32),
            jax.ShapeDtypeStruct((4096, N), jnp.float32),
            jax.ShapeDtypeStruct((2048, N), jnp.float32),
            jax.ShapeDtypeStruct((1024, N), jnp.float32),
        ),
        in_specs=[
            pl.BlockSpec(memory_space=pl.ANY),
            pl.BlockSpec(memory_space=pltpu.VMEM),
        ],
        out_specs=(pl.BlockSpec(memory_space=pl.ANY),) * 4,
        scratch_shapes=[
            pltpu.VMEM((TM, K), jnp.float32),
            pltpu.VMEM((TM, N), jnp.float32),
            pltpu.VMEM((TM, N), jnp.float32),
            pltpu.SemaphoreType.DMA((2,)),
            pltpu.SemaphoreType.DMA((6,)),
            pltpu.SemaphoreType.DMA((6,)),
        ],
        compiler_params=pltpu.CompilerParams(
            collective_id=0, vmem_limit_bytes=60 * 1024 * 1024),
    )(x, w_mat)
    return outs[0]
